# baseline (device time: 12867 ns/iter reference)
import jax
import jax.numpy as jnp
from jax import lax
from jax.experimental import pallas as pl
from jax.experimental.pallas import tpu as pltpu

N_DEV = 32
GROUP = 8
N_Z = N_DEV // GROUP


def _combine(vals, idxs):
    best_val = jnp.max(vals, axis=0)
    best_idx = jnp.min(
        jnp.where(vals == best_val[None, :], idxs, jnp.float32(1e9)), axis=0
    )
    return best_val, best_idx


def kernel(x):
    m_per, n = x.shape

    def body(x_ref, out_ref, gz_ref, gp_ref, xv_ref, res_ref,
             sendz, recvz, sendp, recvp, copy_sem, out_sem):
        my_pos = lax.axis_index("i")
        my_rank = lax.rem(my_pos, GROUP)
        group_base = my_pos - my_rank

        in_copy = pltpu.make_async_copy(x_ref, xv_ref, copy_sem)
        in_copy.start()

        barrier_sem = pltpu.get_barrier_semaphore()
        z_partners = []
        for dz in range(1, N_Z):
            p = lax.rem(my_pos + GROUP * dz, N_DEV)
            z_partners.append(p)
            pl.semaphore_signal(
                barrier_sem, inc=8,
                device_id=(p,), device_id_type=pl.DeviceIdType.MESH,
            )
        p_partners = []
        for k in range(1, GROUP):
            p = group_base + lax.rem(my_rank + k, GROUP)
            p_partners.append(p)
            pl.semaphore_signal(
                barrier_sem, inc=1,
                device_id=(p,), device_id_type=pl.DeviceIdType.MESH,
            )

        in_copy.wait()

        xv = xv_ref[:, :]
        val = jnp.max(xv, axis=0)
        rows = lax.broadcasted_iota(jnp.int32, (m_per, n), 0)
        loc_idx = jnp.min(jnp.where(xv == val[None, :], rows, m_per), axis=0)
        gidx = (loc_idx + my_pos * m_per).astype(jnp.float32)
        gz_ref[0, 0, :] = val
        gz_ref[1, 0, :] = gidx

        pl.semaphore_wait(barrier_sem, 24)
        rdmas_z = []
        for dz, p in zip(range(1, N_Z), z_partners):
            rdma = pltpu.make_async_remote_copy(
                src_ref=gz_ref.at[:, 0, :],
                dst_ref=gz_ref.at[:, dz, :],
                send_sem=sendz.at[dz],
                recv_sem=recvz.at[dz],
                device_id=(p,),
                device_id_type=pl.DeviceIdType.MESH,
            )
            rdma.start()
            rdmas_z.append(rdma)
        for rdma in rdmas_z:
            rdma.wait_recv()
        cval, cidx = _combine(gz_ref[0, :, :], gz_ref[1, :, :])
        gp_ref[0, 0, :] = cval
        gp_ref[1, 0, :] = cidx

        pl.semaphore_wait(barrier_sem, GROUP - 1)
        rdmas_p = []
        for k, p in zip(range(1, GROUP), p_partners):
            rdma = pltpu.make_async_remote_copy(
                src_ref=gp_ref.at[:, 0, :],
                dst_ref=gp_ref.at[:, k, :],
                send_sem=sendp.at[k],
                recv_sem=recvp.at[k],
                device_id=(p,),
                device_id_type=pl.DeviceIdType.MESH,
            )
            rdma.start()
            rdmas_p.append(rdma)
        for rdma in rdmas_p:
            rdma.wait_recv()
        best_val, best_idx = _combine(gp_ref[0, :, :], gp_ref[1, :, :])
        res_ref[0, :] = best_val
        res_ref[1, :] = best_idx

        out_copy = pltpu.make_async_copy(res_ref, out_ref, out_sem)
        out_copy.start()
        out_copy.wait()

        for rdma in rdmas_z:
            rdma.wait_send()
        for rdma in rdmas_p:
            rdma.wait_send()

    out_shape = jax.ShapeDtypeStruct((2, n), jnp.float32)
    return pl.pallas_call(
        body,
        out_shape=out_shape,
        in_specs=[pl.BlockSpec(memory_space=pl.ANY)],
        out_specs=pl.BlockSpec(memory_space=pl.ANY),
        scratch_shapes=[
            pltpu.VMEM((2, N_Z, n), jnp.float32),
            pltpu.VMEM((2, GROUP, n), jnp.float32),
            pltpu.VMEM((m_per, n), jnp.float32),
            pltpu.VMEM((2, n), jnp.float32),
            pltpu.SemaphoreType.DMA((N_Z,)),
            pltpu.SemaphoreType.DMA((N_Z,)),
            pltpu.SemaphoreType.DMA((GROUP,)),
            pltpu.SemaphoreType.DMA((GROUP,)),
            pltpu.SemaphoreType.DMA,
            pltpu.SemaphoreType.DMA,
        ],
        compiler_params=pltpu.CompilerParams(collective_id=0),
    )(x)


# device time: 12830 ns/iter; 1.0029x vs baseline; 1.0029x over previous
import jax
import jax.numpy as jnp
from jax import lax
from jax.experimental import pallas as pl
from jax.experimental.pallas import tpu as pltpu

N_DEV = 32
GROUP = 8
N_Z = N_DEV // GROUP


def _combine(vals, idxs):
    best_val = jnp.max(vals, axis=0)
    best_idx = jnp.min(
        jnp.where(vals == best_val[None, :], idxs, jnp.float32(1e9)), axis=0
    )
    return best_val, best_idx


def kernel(x):
    m_per, n = x.shape

    def body(x_ref, out_ref, gz_ref, gp_ref, xv_ref, res_ref,
             sendz, recvz, sendp, recvp, copy_sem, out_sem):
        my_pos = lax.axis_index("i")
        my_rank = lax.rem(my_pos, GROUP)
        group_base = my_pos - my_rank

        in_copy = pltpu.make_async_copy(x_ref, xv_ref, copy_sem)
        in_copy.start()

        barrier_sem = pltpu.get_barrier_semaphore()
        z_partners = []
        for dz in range(1, N_Z):
            p = lax.rem(my_pos + GROUP * dz, N_DEV)
            z_partners.append(p)
            pl.semaphore_signal(
                barrier_sem, inc=8,
                device_id=(p,), device_id_type=pl.DeviceIdType.MESH,
            )
        p_partners = []
        for k in range(1, GROUP):
            p = group_base + lax.rem(my_rank + k, GROUP)
            p_partners.append(p)
            pl.semaphore_signal(
                barrier_sem, inc=1,
                device_id=(p,), device_id_type=pl.DeviceIdType.MESH,
            )

        in_copy.wait()

        xv = xv_ref[:, :]
        val = jnp.max(xv, axis=0)
        rows = lax.broadcasted_iota(jnp.int32, (m_per, n), 0)
        loc_idx = jnp.min(jnp.where(xv == val[None, :], rows, m_per), axis=0)
        gidx = (loc_idx + my_pos * m_per).astype(jnp.float32)
        gz_ref[0, 0, :] = val
        gz_ref[1, 0, :] = gidx

        pl.semaphore_wait(barrier_sem, 24)
        rdmas_z = []
        for dz, p in zip(range(1, N_Z), z_partners):
            rdma = pltpu.make_async_remote_copy(
                src_ref=gz_ref.at[:, 0, :],
                dst_ref=gz_ref.at[:, dz, :],
                send_sem=sendz.at[dz],
                recv_sem=recvz.at[dz],
                device_id=(p,),
                device_id_type=pl.DeviceIdType.MESH,
            )
            rdma.start()
            rdmas_z.append(rdma)
        for rdma in rdmas_z:
            rdma.wait_recv()

        pl.semaphore_wait(barrier_sem, GROUP - 1)
        rdmas_p = []
        for k, p in zip(range(1, GROUP), p_partners):
            rdma = pltpu.make_async_remote_copy(
                src_ref=gz_ref,
                dst_ref=gp_ref.at[:, k, :, :],
                send_sem=sendp.at[k],
                recv_sem=recvp.at[k],
                device_id=(p,),
                device_id_type=pl.DeviceIdType.MESH,
            )
            rdma.start()
            rdmas_p.append(rdma)
        gp_ref[:, 0, :, :] = gz_ref[:, :, :]
        for rdma in rdmas_p:
            rdma.wait_recv()
        best_val, best_idx = _combine(
            gp_ref[0, :, :, :].reshape(GROUP * N_Z, n),
            gp_ref[1, :, :, :].reshape(GROUP * N_Z, n),
        )
        res_ref[0, :] = best_val
        res_ref[1, :] = best_idx

        out_copy = pltpu.make_async_copy(res_ref, out_ref, out_sem)
        out_copy.start()
        out_copy.wait()

        for rdma in rdmas_z:
            rdma.wait_send()
        for rdma in rdmas_p:
            rdma.wait_send()

    out_shape = jax.ShapeDtypeStruct((2, n), jnp.float32)
    return pl.pallas_call(
        body,
        out_shape=out_shape,
        in_specs=[pl.BlockSpec(memory_space=pl.ANY)],
        out_specs=pl.BlockSpec(memory_space=pl.ANY),
        scratch_shapes=[
            pltpu.VMEM((2, N_Z, n), jnp.float32),
            pltpu.VMEM((2, GROUP, N_Z, n), jnp.float32),
            pltpu.VMEM((m_per, n), jnp.float32),
            pltpu.VMEM((2, n), jnp.float32),
            pltpu.SemaphoreType.DMA((N_Z,)),
            pltpu.SemaphoreType.DMA((N_Z,)),
            pltpu.SemaphoreType.DMA((GROUP,)),
            pltpu.SemaphoreType.DMA((GROUP,)),
            pltpu.SemaphoreType.DMA,
            pltpu.SemaphoreType.DMA,
        ],
        compiler_params=pltpu.CompilerParams(collective_id=0),
    )(x)
